# baseline (speedup 1.0000x reference)
import jax
import jax.numpy as jnp
from jax import lax
from jax.experimental import pallas as pl
from jax.experimental.pallas import tpu as pltpu
from jax.experimental.pallas import tpu_sc as plsc

B, T, H = 16, 4096, 32
BT = 64
NC, NS, L = 2, 16, 16

HT = H // 8
TT = T // 128
NW = NC * NS
VPC = BT // L


BLK_TT = TT // 2


def _compute(buf):
    def tt_body(tt, _):
        for h8 in range(8):
            for chunk in range(2):
                base = chunk * BT
                car = jnp.float32(0.0)
                for k in range(VPC):
                    off = base + k * L
                    v = buf[tt, h8, pl.ds(off, L)]
                    s = plsc.cumsum(v) + car
                    buf[tt, h8, pl.ds(off, L)] = s
                    car = jnp.squeeze(lax.slice(s, (15,), (16,)))
        return 0

    lax.fori_loop(0, BLK_TT, tt_body, 0)


def _body(x_hbm, out_hbm, bufs, sins, souts):
    wid = lax.axis_index("s") * NC + lax.axis_index("c")
    q0 = wid * 2
    blocks = []
    for j in range(4):
        q = q0 + j // 2
        blocks.append((q // HT, q % HT, (j % 2) * BLK_TT))

    d_ins = [
        pltpu.async_copy(
            x_hbm.at[b, ht, pl.ds(t, BLK_TT)], bufs[j], sins[j])
        for j, (b, ht, t) in enumerate(blocks)
    ]
    d_outs = []
    for j, (b, ht, t) in enumerate(blocks):
        d_ins[j].wait()
        _compute(bufs[j])
        d_outs.append(pltpu.async_copy(
            bufs[j], out_hbm.at[b, ht, pl.ds(t, BLK_TT)], souts[j]))
    for d in d_outs:
        d.wait()


@jax.jit
def kernel(g):
    x = g.transpose(0, 2, 1).reshape(B, HT, 8, TT, 128).transpose(0, 1, 3, 2, 4)
    run = pl.kernel(
        _body,
        out_type=jax.ShapeDtypeStruct((B, HT, TT, 8, 128), jnp.float32),
        mesh=plsc.VectorSubcoreMesh(
            core_axis_name="c", subcore_axis_name="s",
            num_cores=NC, num_subcores=NS,
        ),
        scratch_types=[
            [pltpu.VMEM((BLK_TT, 8, 128), jnp.float32)] * 4,
            [pltpu.SemaphoreType.DMA] * 4,
            [pltpu.SemaphoreType.DMA] * 4,
        ],
        compiler_params=pltpu.CompilerParams(needs_layout_passes=False),
    )
    y = run(x)
    return y.transpose(0, 1, 3, 2, 4).reshape(B, H, T).transpose(0, 2, 1)


# speedup vs baseline: 1.0017x; 1.0017x over previous
import jax
import jax.numpy as jnp
from jax import lax
from jax.experimental import pallas as pl
from jax.experimental.pallas import tpu as pltpu
from jax.experimental.pallas import tpu_sc as plsc

B, T, H = 16, 4096, 32
BT = 64
NC, NS, L = 2, 16, 16

HT = H // 8
TT = T // 128
VPC = BT // L
BLK_TT = TT // 2


def _compute(buf):
    def tt_body(tt, _):
        for h8 in range(8):
            for chunk in range(2):
                base = chunk * BT
                car = jnp.float32(0.0)
                for k in range(VPC):
                    off = base + k * L
                    v = buf[tt, h8, pl.ds(off, L)]
                    s = plsc.cumsum(v) + car
                    buf[tt, h8, pl.ds(off, L)] = s
                    car = jnp.squeeze(lax.slice(s, (15,), (16,)))
        return 0

    lax.fori_loop(0, BLK_TT, tt_body, 0)


def _body(x_hbm, out_hbm, bufs, sins, souts):
    wid = lax.axis_index("s") * NC + lax.axis_index("c")
    q0 = wid * 2
    blocks = []
    for j in range(4):
        q = q0 + j // 2
        blocks.append((q // HT, q % HT, (j % 2) * BLK_TT))

    d_ins = [
        pltpu.async_copy(
            x_hbm.at[b, ht, pl.ds(t, BLK_TT)], bufs[j], sins[j])
        for j, (b, ht, t) in enumerate(blocks)
    ]
    d_outs = []
    for j, (b, ht, t) in enumerate(blocks):
        d_ins[j].wait()
        _compute(bufs[j])
        d_outs.append(pltpu.async_copy(
            bufs[j], out_hbm.at[b, ht, pl.ds(t, BLK_TT)], souts[j]))
    for d in d_outs:
        d.wait()


@jax.jit
def kernel(g):
    x = g.transpose(0, 2, 1).reshape(B, HT, 8, TT, 128).transpose(0, 1, 3, 2, 4)
    run = pl.kernel(
        _body,
        out_type=jax.ShapeDtypeStruct((B, HT, TT, 8, 128), jnp.float32),
        mesh=plsc.VectorSubcoreMesh(
            core_axis_name="c", subcore_axis_name="s",
            num_cores=NC, num_subcores=NS,
        ),
        scratch_types=[
            [pltpu.VMEM((BLK_TT, 8, 128), jnp.float32)] * 4,
            [pltpu.SemaphoreType.DMA] * 4,
            [pltpu.SemaphoreType.DMA] * 4,
        ],
        compiler_params=pltpu.CompilerParams(needs_layout_passes=False),
    )
    y = run(x)
    return y.transpose(0, 1, 3, 2, 4).reshape(B, H, T).transpose(0, 2, 1)
